# Initial kernel scaffold; baseline (speedup 1.0000x reference)
import jax
import jax.numpy as jnp
from jax.experimental import pallas as pl


def kernel(edge_view, edge_save, edge_buy, edge_viewed_by, edge_saved_by, edge_bought_by, user_ids, item_ids, user_table, item_table, W1_l, W1_r, b1, W2_l, W2_r, b2, D1, bd1, D2, bd2):
    raise NotImplementedError("write your pallas kernel here")


# speedup vs baseline: 1.0557x; 1.0557x over previous
import functools

import jax
import jax.numpy as jnp
from jax.experimental import pallas as pl

_BLK = 1000
_HBLK = 2048


def _layer_kernel(a0_ref, a1_ref, a2_ref, xdst_ref, wl_ref, wrm_ref, bm_ref,
                  out_ref, *, relu):
    acc = jnp.dot(xdst_ref[...], wrm_ref[...],
                  preferred_element_type=jnp.float32)
    acc += jnp.dot(a0_ref[...], wl_ref[0], preferred_element_type=jnp.float32)
    acc += jnp.dot(a1_ref[...], wl_ref[1], preferred_element_type=jnp.float32)
    acc += jnp.dot(a2_ref[...], wl_ref[2], preferred_element_type=jnp.float32)
    acc += bm_ref[...]
    if relu:
        acc = jnp.maximum(acc, 0.0)
    out_ref[...] = acc


def _layer_call(a0, a1, a2, xdst, wl, wrm, bm, relu):
    n, h = xdst.shape
    row_spec = pl.BlockSpec((_BLK, h), lambda i: (i, 0))
    return pl.pallas_call(
        functools.partial(_layer_kernel, relu=relu),
        grid=(n // _BLK,),
        in_specs=[
            row_spec, row_spec, row_spec, row_spec,
            pl.BlockSpec((3, h, h), lambda i: (0, 0, 0)),
            pl.BlockSpec((h, h), lambda i: (0, 0)),
            pl.BlockSpec((1, h), lambda i: (0, 0)),
        ],
        out_specs=row_spec,
        out_shape=jax.ShapeDtypeStruct((n, h), jnp.float32),
    )(a0, a1, a2, xdst, wl, wrm, bm)


def _head_kernel(ue_ref, ie_ref, d1u_ref, d1i_ref, bd1_ref, d2_ref, bd2_ref,
                 out_ref):
    h = jnp.dot(ue_ref[...], d1u_ref[...], preferred_element_type=jnp.float32)
    h += jnp.dot(ie_ref[...], d1i_ref[...], preferred_element_type=jnp.float32)
    h += bd1_ref[...]
    h = jnp.maximum(h, 0.0)
    out = jnp.dot(h, d2_ref[...], preferred_element_type=jnp.float32)
    out_ref[...] = out + bd2_ref[...]


def _head_call(ue, ie, d1u, d1i, bd1, d2p, bd2p):
    b, h = ue.shape
    row_spec = pl.BlockSpec((_HBLK, h), lambda i: (i, 0))
    w_spec = pl.BlockSpec((h, h), lambda i: (0, 0))
    b_spec = pl.BlockSpec((1, h), lambda i: (0, 0))
    return pl.pallas_call(
        _head_kernel,
        grid=(b // _HBLK,),
        in_specs=[row_spec, row_spec, w_spec, w_spec, b_spec, w_spec, b_spec],
        out_specs=row_spec,
        out_shape=jax.ShapeDtypeStruct((b, h), jnp.float32),
    )(ue, ie, d1u, d1i, bd1, d2p, bd2p)


def _agg_mean(x_src, ei, n_dst):
    src = ei[0]
    dst = ei[1]
    msg = jnp.take(x_src, src, axis=0)
    s = jax.ops.segment_sum(msg, dst, num_segments=n_dst)
    c = jax.ops.segment_sum(jnp.ones((ei.shape[1],), x_src.dtype), dst,
                            num_segments=n_dst)
    return s / jnp.maximum(c, 1.0)[:, None]


def kernel(edge_view, edge_save, edge_buy, edge_viewed_by, edge_saved_by,
           edge_bought_by, user_ids, item_ids, user_table, item_table,
           W1_l, W1_r, b1, W2_l, W2_r, b2, D1, bd1, D2, bd2):
    edges = [edge_view, edge_save, edge_buy,
             edge_viewed_by, edge_saved_by, edge_bought_by]
    nu = user_table.shape[0]
    ni = item_table.shape[0]

    def layer(xu, xi, Wl, Wr, b, relu):
        ai = [_agg_mean(xu, edges[r], ni) for r in range(3)]
        au = [_agg_mean(xi, edges[r], nu) for r in range(3, 6)]
        item_out = _layer_call(ai[0], ai[1], ai[2], xi,
                               Wl[0:3] / 3.0,
                               jnp.mean(Wr[0:3], axis=0),
                               jnp.mean(b[0:3], axis=0)[None, :], relu)
        user_out = _layer_call(au[0], au[1], au[2], xu,
                               Wl[3:6] / 3.0,
                               jnp.mean(Wr[3:6], axis=0),
                               jnp.mean(b[3:6], axis=0)[None, :], relu)
        return user_out, item_out

    xu, xi = layer(user_table, item_table, W1_l, W1_r, b1, relu=True)
    xu, xi = layer(xu, xi, W2_l, W2_r, b2, relu=False)

    ue = jnp.take(xu, user_ids, axis=0)
    ie = jnp.take(xi, item_ids, axis=0)

    hid = D1.shape[1]
    d1u = D1[:hid]
    d1i = D1[hid:]
    nout = D2.shape[1]
    d2p = jnp.zeros((hid, hid), jnp.float32).at[:, :nout].set(D2)
    bd2p = jnp.zeros((1, hid), jnp.float32).at[0, :nout].set(bd2)
    logits = _head_call(ue, ie, d1u, d1i, bd1[None, :], d2p, bd2p)
    return logits[:, :nout]
